# baseline (speedup 1.0000x reference)
import functools

import jax
import jax.numpy as jnp
from jax import lax
from jax.experimental import pallas as pl
from jax.experimental.pallas import tpu as pltpu

M = 4096
K = 32
N = 8192
R = 512
NB = M // R
COMMITMENT_COST = 0.25


def _vq_body(x_ref, xsq_ref, wsq_ref, iota_ref, w_ref, enc_ref, q_ref,
             idx_ref, loss_ref, perp_ref, counts_ref, acc_ref):
    step = pl.program_id(0)

    @pl.when(step == 0)
    def _init():
        counts_ref[...] = jnp.zeros_like(counts_ref)
        acc_ref[0] = 0.0

    x = x_ref[...]
    xm2 = x * (-2.0)
    w = w_ref[...]
    xwm2 = lax.dot_general(xm2, w, (((1,), (1,)), ((), ())),
                           preferred_element_type=jnp.float32)
    d = (xsq_ref[...] + wsq_ref[...]) + xwm2
    mval = jnp.min(d, axis=1, keepdims=True)
    d_i = lax.bitcast_convert_type(d, jnp.int32)
    mval_i = lax.bitcast_convert_type(mval, jnp.int32)
    iota_f = iota_ref[...]
    idx_f = jnp.min(jnp.where(d_i == mval_i, iota_f, jnp.float32(N)), axis=1)
    idx_ref[0, 0, :] = idx_f.astype(jnp.int32)

    iota_i = lax.bitcast_convert_type(iota_f, jnp.int32)
    idx_i = lax.bitcast_convert_type(idx_f[:, None], jnp.int32)
    enc = (iota_i == idx_i).astype(jnp.float32)
    enc_ref[...] = enc

    q = lax.dot_general(enc, w, (((1,), (0,)), ((), ())),
                        preferred_element_type=jnp.float32)
    q_ref[...] = x + (q - x)

    ones_r = jnp.ones((1, R), jnp.float32)
    counts_ref[...] += lax.dot_general(ones_r, enc, (((1,), (0,)), ((), ())),
                                       preferred_element_type=jnp.float32)
    acc_ref[0] += jnp.sum((q - x) ** 2)

    @pl.when(step == NB - 1)
    def _fini():
        loss_ref[0, 0] = COMMITMENT_COST * acc_ref[0] / (M * K)
        p = counts_ref[...] * (1.0 / M)
        perp_ref[0, 0] = jnp.exp(-jnp.sum(p * jnp.log(p + 1e-10)))


@functools.partial(jax.jit, static_argnames=("interpret",))
def _vq_call(x_flat, xsq, wsq, iota_row, embedding_weight, interpret=False):
    out_shapes = (
        jax.ShapeDtypeStruct((M, N), jnp.float32),
        jax.ShapeDtypeStruct((M, K), jnp.float32),
        jax.ShapeDtypeStruct((NB, 1, R), jnp.int32),
        jax.ShapeDtypeStruct((1, 1), jnp.float32),
        jax.ShapeDtypeStruct((1, 1), jnp.float32),
    )
    out_specs = (
        pl.BlockSpec((R, N), lambda i: (i, 0)),
        pl.BlockSpec((R, K), lambda i: (i, 0)),
        pl.BlockSpec((1, 1, R), lambda i: (i, 0, 0)),
        pl.BlockSpec(memory_space=pltpu.SMEM),
        pl.BlockSpec(memory_space=pltpu.SMEM),
    )
    in_specs = [
        pl.BlockSpec((R, K), lambda i: (i, 0)),
        pl.BlockSpec((R, 1), lambda i: (i, 0)),
        pl.BlockSpec((1, N), lambda i: (0, 0)),
        pl.BlockSpec((1, N), lambda i: (0, 0)),
        pl.BlockSpec((N, K), lambda i: (0, 0)),
    ]
    return pl.pallas_call(
        _vq_body,
        grid=(NB,),
        in_specs=in_specs,
        out_specs=out_specs,
        out_shape=out_shapes,
        scratch_shapes=[
            pltpu.VMEM((1, N), jnp.float32),
            pltpu.SMEM((1,), jnp.float32),
        ],
        compiler_params=pltpu.CompilerParams(vmem_limit_bytes=128 * 1024 * 1024),
        interpret=interpret,
    )(x_flat, xsq, wsq, iota_row, embedding_weight)


def kernel(inputs, embedding_weight, interpret=False):
    x_flat = jnp.transpose(inputs, (0, 2, 3, 1)).reshape(M, K)
    xsq = jnp.sum(x_flat ** 2, axis=1, keepdims=True)
    wsq = jnp.sum(embedding_weight ** 2, axis=1).reshape(1, N)
    iota_row = jnp.arange(N, dtype=jnp.float32).reshape(1, N)
    enc, q, idx, loss, perp = _vq_call(x_flat, xsq, wsq, iota_row,
                                       embedding_weight, interpret=interpret)
    quantized_out = jnp.transpose(q.reshape(16, 16, 16, K), (0, 3, 1, 2))
    return (quantized_out,
            loss.reshape(()),
            perp.reshape(()),
            idx.reshape(M, 1),
            enc)
